# baseline (speedup 1.0000x reference)
import jax
import jax.numpy as jnp
from jax.experimental import pallas as pl

N = 10000
L = 3


def kernel(x, edge_index, batch, W1, b1, W2, b2, gamma, beta, fc1_W, fc1_b, fc2_W, fc2_b):
    src = edge_index[0]
    dst = edge_index[1]
    h = x
    xs = []
    for i in range(L):
        agg = jax.ops.segment_sum(h[src], dst, num_segments=N)
        z = h + agg
        z = jnp.maximum(z @ W1[i] + b1[i], 0.0) @ W2[i] + b2[i]
        z = jnp.maximum(z, 0.0)
        mean = jnp.mean(z, axis=0)
        var = jnp.var(z, axis=0)
        z = (z - mean) / jnp.sqrt(var + 1e-5) * gamma[i] + beta[i]
        xs.append(z)
        h = z
    pooled = [jax.ops.segment_sum(z, batch, num_segments=1) for z in xs]
    g = jnp.concatenate(pooled, axis=1)
    g = jnp.maximum(g @ fc1_W + fc1_b, 0.0)
    out = g @ fc2_W + fc2_b
    return out[0]


# speedup vs baseline: 1.5184x; 1.5184x over previous
import functools

import jax
import jax.numpy as jnp
from jax import lax
from jax.experimental import pallas as pl
from jax.experimental.pallas import tpu as pltpu
from jax.experimental.pallas import tpu_sc as plsc

N = 10000
E = 320000
D = 128
L = 3
B = 80
NW = 32
CB = E // (B * NW)

_mesh = plsc.VectorSubcoreMesh(core_axis_name="c", subcore_axis_name="s")


@functools.partial(
    pl.kernel,
    out_type=jax.ShapeDtypeStruct((E, D), jnp.float32),
    mesh=_mesh,
    scratch_types=[
        pltpu.VMEM((CB, B), jnp.int32),
        pltpu.VMEM((B, D), jnp.float32),
        pltpu.SemaphoreType.DMA,
    ],
)
def _sc_gather(h_hbm, src_hbm, out_hbm, idx_v, rows_v, sem):
    wid = lax.axis_index("s") * 2 + lax.axis_index("c")
    pltpu.sync_copy(src_hbm.at[wid], idx_v)

    def body(j, carry):
        g = wid * CB + j
        pltpu.async_copy(h_hbm.at[idx_v.at[j]], rows_v, sem).wait()
        pltpu.sync_copy(rows_v, out_hbm.at[pl.ds(g * B, B)])
        return carry

    lax.fori_loop(0, CB, body, 0)


def kernel(x, edge_index, batch, W1, b1, W2, b2, gamma, beta, fc1_W, fc1_b, fc2_W, fc2_b):
    src = edge_index[0]
    dst = edge_index[1]
    src3d = src.reshape(NW, CB, B)
    h = x
    xs = []
    for i in range(L):
        rows = _sc_gather(h, src3d)
        agg = jax.ops.segment_sum(rows, dst, num_segments=N)
        z = h + agg
        z = jnp.maximum(z @ W1[i] + b1[i], 0.0) @ W2[i] + b2[i]
        z = jnp.maximum(z, 0.0)
        mean = jnp.mean(z, axis=0)
        var = jnp.var(z, axis=0)
        z = (z - mean) / jnp.sqrt(var + 1e-5) * gamma[i] + beta[i]
        xs.append(z)
        h = z
    pooled = [jax.ops.segment_sum(z, batch, num_segments=1) for z in xs]
    g = jnp.concatenate(pooled, axis=1)
    g = jnp.maximum(g @ fc1_W + fc1_b, 0.0)
    out = g @ fc2_W + fc2_b
    return out[0]
